# Initial kernel scaffold; baseline (speedup 1.0000x reference)
import jax
import jax.numpy as jnp
from jax.experimental import pallas as pl


def kernel(glyphs, blstats, inv_glyphs, emb_table):
    raise NotImplementedError("write your pallas kernel here")


# speedup vs baseline: 2.2296x; 2.2296x over previous
import functools

import jax
import jax.numpy as jnp
from jax import lax
from jax.experimental import pallas as pl
from jax.experimental.pallas import tpu as pltpu
from jax.experimental.pallas import tpu_sc as plsc

_MAXG = 5976
_T, _B, _R, _C, _D, _NINV = 16, 32, 21, 79, 64, 55
_NTB = _T * _B
_RC = _R * _C
_RCP = 1664
_META0 = _RCP
_INVO = _RCP + 16
_PACKW = _INVO + 64
_NC, _NS = 2, 16
_DPT = 8
_NDB = _D // _DPT
_ETW = 5984
_SEG = _DPT * _RC
_NFULL = _RC // 16
_TAIL = _RC - 16 * _NFULL
_VIC0 = 0
_SELF0 = _DPT * 9
_INV0 = _SELF0 + _DPT
_SMW = _INV0 + _DPT * _NINV
_SMBUF = 536


def _sc_body(pack_hbm, et_hbm, screen_hbm, small_hbm, in_v, et_v, sbuf, small_v):
    cid = lax.axis_index("c")
    tid = lax.axis_index("s")
    dblk = tid % _NDB
    grp = tid // _NDB
    ntb_tile = _NTB // (_NC * 2)
    tb0 = (cid * 2 + grp) * ntb_tile
    lane = lax.iota(jnp.int32, 16)

    pltpu.sync_copy(et_hbm.at[pl.ds(dblk * _DPT * _ETW, _DPT * _ETW)], et_v)

    def tb_body(i, carry):
        tb = tb0 + i
        pltpu.sync_copy(pack_hbm.at[tb], in_v)

        def chunk(c, carry2):
            g = in_v[pl.ds(c * 16, 16)]
            for dl in range(_DPT):
                v = plsc.load_gather(et_v, [g + dl * _ETW])
                sbuf[pl.ds(dl * _RC + c * 16, 16)] = v
            return carry2

        lax.fori_loop(0, _NFULL, chunk, 0)

        gt = in_v[pl.ds(_NFULL * 16, 16)]
        tmask = lane < _TAIL
        for dl in range(_DPT):
            v = plsc.load_gather(et_v, [gt + dl * _ETW])
            plsc.store_scatter(
                sbuf, [lane + (dl * _RC + _NFULL * 16)], v, mask=tmask
            )

        xv = plsc.load_gather(in_v, [jnp.full((16,), _META0, jnp.int32)])
        yv = plsc.load_gather(in_v, [jnp.full((16,), _META0 + 1, jnp.int32)])
        ii = lane // 3
        jj = lane - ii * 3
        rr = yv - 1 + ii
        cc = xv - 1 + jj
        inb = (rr >= 0) & (rr < _R) & (cc >= 0) & (cc < _C) & (lane < 9)
        flat = jnp.where(inb, rr * _C + cc, 0)
        gl = plsc.load_gather(in_v, [flat])
        vic = jnp.where(inb, gl, _MAXG)

        vic_vals = []
        for dl in range(_DPT):
            vv = plsc.load_gather(et_v, [vic + dl * _ETW])
            vic_vals.append(vv)
            small_v[pl.ds(_VIC0 + dl * 9, 16)] = vv
        for dl in range(_DPT):
            plsc.store_scatter(
                small_v,
                [jnp.full((16,), _SELF0 + dl, jnp.int32)],
                vic_vals[dl],
                mask=lane == 4,
            )

        ivs = [in_v[pl.ds(_INVO + c2 * 16, 16)] for c2 in range(4)]
        for dl in range(_DPT):
            for c2 in range(4):
                vv = plsc.load_gather(et_v, [ivs[c2] + dl * _ETW])
                small_v[pl.ds(_INV0 + dl * _NINV + c2 * 16, 16)] = vv

        pltpu.sync_copy(sbuf.at[pl.ds(0, _SEG)], screen_hbm.at[tb, dblk])
        pltpu.sync_copy(small_v.at[pl.ds(0, _SMW)], small_hbm.at[tb, dblk])
        return carry

    lax.fori_loop(0, ntb_tile, tb_body, 0)


def kernel(glyphs, blstats, inv_glyphs, emb_table):
    gly = glyphs.reshape(_NTB, _RC).astype(jnp.int32)
    bl = blstats.reshape(_NTB, blstats.shape[-1]).astype(jnp.int32)
    inv = inv_glyphs.reshape(_NTB, _NINV).astype(jnp.int32)

    pack = jnp.full((_NTB, _PACKW), _MAXG, jnp.int32)
    pack = pack.at[:, :_RC].set(gly)
    pack = pack.at[:, _META0].set(bl[:, 0])
    pack = pack.at[:, _META0 + 1].set(bl[:, 1])
    pack = pack.at[:, _INVO : _INVO + _NINV].set(inv)

    et = jnp.zeros((_D, _ETW), jnp.float32)
    et = et.at[:, : _MAXG + 1].set(emb_table.astype(jnp.float32).T)
    et = et.reshape(_D * _ETW)

    mesh = plsc.VectorSubcoreMesh(
        core_axis_name="c", subcore_axis_name="s", num_cores=_NC, num_subcores=_NS
    )
    run = functools.partial(
        pl.kernel,
        out_type=[
            jax.ShapeDtypeStruct((_NTB, _NDB, _SEG), jnp.float32),
            jax.ShapeDtypeStruct((_NTB, _NDB, _SMW), jnp.float32),
        ],
        mesh=mesh,
        compiler_params=pltpu.CompilerParams(
            needs_layout_passes=False, use_tc_tiling_on_sc=False
        ),
        scratch_types=[
            pltpu.VMEM((_PACKW,), jnp.int32),
            pltpu.VMEM((_DPT * _ETW,), jnp.float32),
            pltpu.VMEM((_SEG + 8,), jnp.float32),
            pltpu.VMEM((_SMBUF,), jnp.float32),
        ],
    )(_sc_body)
    screen_f, small_f = run(pack, et)

    screen = screen_f.reshape(_T, _B, _D, _R, _C)
    vicinity = small_f[:, :, :_SELF0].reshape(_T, _B, _D, 3, 3)
    self_ = small_f[:, :, _SELF0:_INV0].reshape(_T, _B, _D)
    inventory = small_f[:, :, _INV0:].reshape(_T, _B, _D, _NINV)
    return screen, vicinity, inventory, self_


# speedup vs baseline: 2.5190x; 1.1298x over previous
import functools

import jax
import jax.numpy as jnp
from jax import lax
from jax.experimental import pallas as pl
from jax.experimental.pallas import tpu as pltpu
from jax.experimental.pallas import tpu_sc as plsc

_MAXG = 5976
_T, _B, _R, _C, _D, _NINV = 16, 32, 21, 79, 64, 55
_NTB = _T * _B
_RC = _R * _C
_RCP = 1664
_META0 = _RCP
_INVO = _RCP + 16
_PACKW = _INVO + 64
_NC, _NS = 2, 16
_DPT = 8
_NDB = _D // _DPT
_ETW = 5984
_SEG = _DPT * _RC
_NFULL = _RC // 16
_TAIL = _RC - 16 * _NFULL
_VIC0 = 0
_SELF0 = _DPT * 9
_INV0 = _SELF0 + _DPT
_SMW = _INV0 + _DPT * _NINV
_SMBUF = 536
_NTILE = _NTB // (_NC * 2)


def _compute_tb(lane, in_v, et_v, sbuf, small_v):
    def chunk(c, carry2):
        g = in_v[pl.ds(c * 16, 16)]
        for dl in range(_DPT):
            v = plsc.load_gather(et_v, [g + dl * _ETW])
            sbuf[pl.ds(dl * _RC + c * 16, 16)] = v
        return carry2

    lax.fori_loop(0, _NFULL, chunk, 0)

    gt = in_v[pl.ds(_NFULL * 16, 16)]
    tmask = lane < _TAIL
    for dl in range(_DPT):
        v = plsc.load_gather(et_v, [gt + dl * _ETW])
        plsc.store_scatter(sbuf, [lane + (dl * _RC + _NFULL * 16)], v, mask=tmask)

    xv = plsc.load_gather(in_v, [jnp.full((16,), _META0, jnp.int32)])
    yv = plsc.load_gather(in_v, [jnp.full((16,), _META0 + 1, jnp.int32)])
    ii = lane // 3
    jj = lane - ii * 3
    rr = yv - 1 + ii
    cc = xv - 1 + jj
    inb = (rr >= 0) & (rr < _R) & (cc >= 0) & (cc < _C) & (lane < 9)
    flat = jnp.where(inb, rr * _C + cc, 0)
    gl = plsc.load_gather(in_v, [flat])
    vic = jnp.where(inb, gl, _MAXG)

    vic_vals = []
    for dl in range(_DPT):
        vv = plsc.load_gather(et_v, [vic + dl * _ETW])
        vic_vals.append(vv)
        small_v[pl.ds(_VIC0 + dl * 9, 16)] = vv
    for dl in range(_DPT):
        plsc.store_scatter(
            small_v,
            [jnp.full((16,), _SELF0 + dl, jnp.int32)],
            vic_vals[dl],
            mask=lane == 4,
        )

    ivs = [in_v[pl.ds(_INVO + c2 * 16, 16)] for c2 in range(4)]
    for dl in range(_DPT):
        for c2 in range(4):
            vv = plsc.load_gather(et_v, [ivs[c2] + dl * _ETW])
            small_v[pl.ds(_INV0 + dl * _NINV + c2 * 16, 16)] = vv


def _sc_body(
    pack_hbm, et_hbm, screen_hbm, small_hbm,
    in_a, in_b, et_v, sb_a, sb_b, sm_a, sm_b,
    si_a, si_b, ss_a, ss_b, sq_a, sq_b,
):
    cid = lax.axis_index("c")
    tid = lax.axis_index("s")
    dblk = tid % _NDB
    grp = tid // _NDB
    tb0 = (cid * 2 + grp) * _NTILE
    lane = lax.iota(jnp.int32, 16)

    pltpu.sync_copy(et_hbm.at[pl.ds(dblk * _DPT * _ETW, _DPT * _ETW)], et_v)

    bufs = ((in_a, sb_a, sm_a, si_a, ss_a, sq_a), (in_b, sb_b, sm_b, si_b, ss_b, sq_b))

    pltpu.async_copy(pack_hbm.at[tb0], in_a, si_a)
    pltpu.async_copy(pack_hbm.at[tb0 + 1], in_b, si_b)

    @pl.loop(0, _NTILE, step=2)
    def pair(i):
        for b in range(2):
            in_v, sbuf, small_v, si, ss, sq = bufs[b]
            g = i + b
            tb = tb0 + g
            pltpu.make_async_copy(pack_hbm.at[tb], in_v, si).wait()

            @pl.when(g >= 2)
            def _wait_prev_out():
                pltpu.make_async_copy(
                    sbuf.at[pl.ds(0, _SEG)], screen_hbm.at[tb - 2, dblk], ss
                ).wait()
                pltpu.make_async_copy(
                    small_v.at[pl.ds(0, _SMW)], small_hbm.at[tb - 2, dblk], sq
                ).wait()

            _compute_tb(lane, in_v, et_v, sbuf, small_v)

            pltpu.async_copy(sbuf.at[pl.ds(0, _SEG)], screen_hbm.at[tb, dblk], ss)
            pltpu.async_copy(small_v.at[pl.ds(0, _SMW)], small_hbm.at[tb, dblk], sq)

            @pl.when(g + 2 < _NTILE)
            def _prefetch_next():
                pltpu.async_copy(pack_hbm.at[tb + 2], in_v, si)

    for b in range(2):
        in_v, sbuf, small_v, si, ss, sq = bufs[b]
        tb = tb0 + _NTILE - 2 + b
        pltpu.make_async_copy(
            sbuf.at[pl.ds(0, _SEG)], screen_hbm.at[tb, dblk], ss
        ).wait()
        pltpu.make_async_copy(
            small_v.at[pl.ds(0, _SMW)], small_hbm.at[tb, dblk], sq
        ).wait()


def kernel(glyphs, blstats, inv_glyphs, emb_table):
    gly = glyphs.reshape(_NTB, _RC).astype(jnp.int32)
    bl = blstats.reshape(_NTB, blstats.shape[-1]).astype(jnp.int32)
    inv = inv_glyphs.reshape(_NTB, _NINV).astype(jnp.int32)

    pack = jnp.full((_NTB, _PACKW), _MAXG, jnp.int32)
    pack = pack.at[:, :_RC].set(gly)
    pack = pack.at[:, _META0].set(bl[:, 0])
    pack = pack.at[:, _META0 + 1].set(bl[:, 1])
    pack = pack.at[:, _INVO : _INVO + _NINV].set(inv)

    et = jnp.zeros((_D, _ETW), jnp.float32)
    et = et.at[:, : _MAXG + 1].set(emb_table.astype(jnp.float32).T)
    et = et.reshape(_D * _ETW)

    mesh = plsc.VectorSubcoreMesh(
        core_axis_name="c", subcore_axis_name="s", num_cores=_NC, num_subcores=_NS
    )
    run = functools.partial(
        pl.kernel,
        out_type=[
            jax.ShapeDtypeStruct((_NTB, _NDB, _SEG), jnp.float32),
            jax.ShapeDtypeStruct((_NTB, _NDB, _SMW), jnp.float32),
        ],
        mesh=mesh,
        compiler_params=pltpu.CompilerParams(
            needs_layout_passes=False, use_tc_tiling_on_sc=False
        ),
        scratch_types=[
            pltpu.VMEM((_PACKW,), jnp.int32),
            pltpu.VMEM((_PACKW,), jnp.int32),
            pltpu.VMEM((_DPT * _ETW,), jnp.float32),
            pltpu.VMEM((_SEG + 8,), jnp.float32),
            pltpu.VMEM((_SEG + 8,), jnp.float32),
            pltpu.VMEM((_SMBUF,), jnp.float32),
            pltpu.VMEM((_SMBUF,), jnp.float32),
            pltpu.SemaphoreType.DMA,
            pltpu.SemaphoreType.DMA,
            pltpu.SemaphoreType.DMA,
            pltpu.SemaphoreType.DMA,
            pltpu.SemaphoreType.DMA,
            pltpu.SemaphoreType.DMA,
        ],
    )(_sc_body)
    screen_f, small_f = run(pack, et)

    screen = screen_f.reshape(_T, _B, _D, _R, _C)
    vicinity = small_f[:, :, :_SELF0].reshape(_T, _B, _D, 3, 3)
    self_ = small_f[:, :, _SELF0:_INV0].reshape(_T, _B, _D)
    inventory = small_f[:, :, _INV0:].reshape(_T, _B, _D, _NINV)
    return screen, vicinity, inventory, self_


# speedup vs baseline: 3.3659x; 1.3362x over previous
import functools

import jax
import jax.numpy as jnp
from jax import lax
from jax.experimental import pallas as pl
from jax.experimental.pallas import tpu as pltpu
from jax.experimental.pallas import tpu_sc as plsc

_MAXG = 5976
_T, _B, _R, _C, _D, _NINV = 16, 32, 21, 79, 64, 55
_NTB = _T * _B
_RC = _R * _C
_RCP = 1664
_META0 = _RCP
_INVO = _RCP + 16
_PACKW = _INVO + 64
_NC, _NS = 2, 16
_DPT = 8
_NDB = _D // _DPT
_ETW = 5984
_SEG = _DPT * _RC
_NFULL = _RC // 16
_TAIL = _RC - 16 * _NFULL
_VIC0 = 0
_SELF0 = _DPT * 9
_INV0 = _SELF0 + _DPT
_SMW = _INV0 + _DPT * _NINV
_SMBUF = 536
_NTILE = _NTB // (_NC * 2)


def _compute_tb(lane, in_v, et_v, sbuf, small_v):
    @plsc.parallel_loop(0, _NFULL, unroll=4)
    def chunk(c):
        g = in_v[pl.ds(c * 16, 16)]
        for dl in range(_DPT):
            v = plsc.load_gather(et_v, [g + dl * _ETW])
            sbuf[pl.ds(dl * _RC + c * 16, 16)] = v

    gt = in_v[pl.ds(_NFULL * 16, 16)]
    tmask = lane < _TAIL
    for dl in range(_DPT):
        v = plsc.load_gather(et_v, [gt + dl * _ETW])
        plsc.store_scatter(sbuf, [lane + (dl * _RC + _NFULL * 16)], v, mask=tmask)

    xv = plsc.load_gather(in_v, [jnp.full((16,), _META0, jnp.int32)])
    yv = plsc.load_gather(in_v, [jnp.full((16,), _META0 + 1, jnp.int32)])
    ii = lane // 3
    jj = lane - ii * 3
    rr = yv - 1 + ii
    cc = xv - 1 + jj
    inb = (rr >= 0) & (rr < _R) & (cc >= 0) & (cc < _C) & (lane < 9)
    flat = jnp.where(inb, rr * _C + cc, 0)
    gl = plsc.load_gather(in_v, [flat])
    vic = jnp.where(inb, gl, _MAXG)

    vic_vals = []
    for dl in range(_DPT):
        vv = plsc.load_gather(et_v, [vic + dl * _ETW])
        vic_vals.append(vv)
        small_v[pl.ds(_VIC0 + dl * 9, 16)] = vv
    for dl in range(_DPT):
        plsc.store_scatter(
            small_v,
            [jnp.full((16,), _SELF0 + dl, jnp.int32)],
            vic_vals[dl],
            mask=lane == 4,
        )

    ivs = [in_v[pl.ds(_INVO + c2 * 16, 16)] for c2 in range(4)]
    for dl in range(_DPT):
        for c2 in range(4):
            vv = plsc.load_gather(et_v, [ivs[c2] + dl * _ETW])
            small_v[pl.ds(_INV0 + dl * _NINV + c2 * 16, 16)] = vv


def _sc_body(
    pack_hbm, et_hbm, screen_hbm, small_hbm,
    in_a, in_b, et_v, sb_a, sb_b, sm_a, sm_b,
    si_a, si_b, ss_a, ss_b, sq_a, sq_b,
):
    cid = lax.axis_index("c")
    tid = lax.axis_index("s")
    dblk = tid % _NDB
    grp = tid // _NDB
    tb0 = (cid * 2 + grp) * _NTILE
    lane = lax.iota(jnp.int32, 16)

    pltpu.sync_copy(et_hbm.at[pl.ds(dblk * _DPT * _ETW, _DPT * _ETW)], et_v)

    bufs = ((in_a, sb_a, sm_a, si_a, ss_a, sq_a), (in_b, sb_b, sm_b, si_b, ss_b, sq_b))

    pltpu.async_copy(pack_hbm.at[tb0], in_a, si_a)
    pltpu.async_copy(pack_hbm.at[tb0 + 1], in_b, si_b)

    @pl.loop(0, _NTILE, step=2)
    def pair(i):
        for b in range(2):
            in_v, sbuf, small_v, si, ss, sq = bufs[b]
            g = i + b
            tb = tb0 + g
            pltpu.make_async_copy(pack_hbm.at[tb], in_v, si).wait()

            @pl.when(g >= 2)
            def _wait_prev_out():
                pltpu.make_async_copy(
                    sbuf.at[pl.ds(0, _SEG)], screen_hbm.at[tb - 2, dblk], ss
                ).wait()
                pltpu.make_async_copy(
                    small_v.at[pl.ds(0, _SMW)], small_hbm.at[tb - 2, dblk], sq
                ).wait()

            _compute_tb(lane, in_v, et_v, sbuf, small_v)

            pltpu.async_copy(sbuf.at[pl.ds(0, _SEG)], screen_hbm.at[tb, dblk], ss)
            pltpu.async_copy(small_v.at[pl.ds(0, _SMW)], small_hbm.at[tb, dblk], sq)

            @pl.when(g + 2 < _NTILE)
            def _prefetch_next():
                pltpu.async_copy(pack_hbm.at[tb + 2], in_v, si)

    for b in range(2):
        in_v, sbuf, small_v, si, ss, sq = bufs[b]
        tb = tb0 + _NTILE - 2 + b
        pltpu.make_async_copy(
            sbuf.at[pl.ds(0, _SEG)], screen_hbm.at[tb, dblk], ss
        ).wait()
        pltpu.make_async_copy(
            small_v.at[pl.ds(0, _SMW)], small_hbm.at[tb, dblk], sq
        ).wait()


def kernel(glyphs, blstats, inv_glyphs, emb_table):
    gly = glyphs.reshape(_NTB, _RC).astype(jnp.int32)
    bl = blstats.reshape(_NTB, blstats.shape[-1]).astype(jnp.int32)
    inv = inv_glyphs.reshape(_NTB, _NINV).astype(jnp.int32)

    pack = jnp.full((_NTB, _PACKW), _MAXG, jnp.int32)
    pack = pack.at[:, :_RC].set(gly)
    pack = pack.at[:, _META0].set(bl[:, 0])
    pack = pack.at[:, _META0 + 1].set(bl[:, 1])
    pack = pack.at[:, _INVO : _INVO + _NINV].set(inv)

    et = jnp.zeros((_D, _ETW), jnp.float32)
    et = et.at[:, : _MAXG + 1].set(emb_table.astype(jnp.float32).T)
    et = et.reshape(_D * _ETW)

    mesh = plsc.VectorSubcoreMesh(
        core_axis_name="c", subcore_axis_name="s", num_cores=_NC, num_subcores=_NS
    )
    run = functools.partial(
        pl.kernel,
        out_type=[
            jax.ShapeDtypeStruct((_NTB, _NDB, _SEG), jnp.float32),
            jax.ShapeDtypeStruct((_NTB, _NDB, _SMW), jnp.float32),
        ],
        mesh=mesh,
        compiler_params=pltpu.CompilerParams(
            needs_layout_passes=False, use_tc_tiling_on_sc=False
        ),
        scratch_types=[
            pltpu.VMEM((_PACKW,), jnp.int32),
            pltpu.VMEM((_PACKW,), jnp.int32),
            pltpu.VMEM((_DPT * _ETW,), jnp.float32),
            pltpu.VMEM((_SEG + 8,), jnp.float32),
            pltpu.VMEM((_SEG + 8,), jnp.float32),
            pltpu.VMEM((_SMBUF,), jnp.float32),
            pltpu.VMEM((_SMBUF,), jnp.float32),
            pltpu.SemaphoreType.DMA,
            pltpu.SemaphoreType.DMA,
            pltpu.SemaphoreType.DMA,
            pltpu.SemaphoreType.DMA,
            pltpu.SemaphoreType.DMA,
            pltpu.SemaphoreType.DMA,
        ],
    )(_sc_body)
    screen_f, small_f = run(pack, et)

    screen = screen_f.reshape(_T, _B, _D, _R, _C)
    vicinity = small_f[:, :, :_SELF0].reshape(_T, _B, _D, 3, 3)
    self_ = small_f[:, :, _SELF0:_INV0].reshape(_T, _B, _D)
    inventory = small_f[:, :, _INV0:].reshape(_T, _B, _D, _NINV)
    return screen, vicinity, inventory, self_
